# baseline (speedup 1.0000x reference)
import functools

import jax
import jax.numpy as jnp
from jax import lax
from jax.experimental import pallas as pl
from jax.experimental.pallas import tpu as pltpu
from jax.experimental.pallas import tpu_sc as plsc

IMAGE_TOKEN = 32000
_NUM_CORES = 2
_NUM_SUBCORES = 16
_NW = _NUM_CORES * _NUM_SUBCORES
_BLK = 8


def _mask_pos_kernel(ids_ref, mask_ref, outmask_ref, pos_ref, pvec_ref,
                     de_ref, di_ref, *, S, P, E):
    B = ids_ref.shape[0]
    lane_e = lax.broadcasted_iota(jnp.int32, (1, E), 1)
    lane_s = lax.broadcasted_iota(jnp.int32, (1, S), 1)
    lane_p = lax.broadcasted_iota(jnp.int32, (1, P), 1)
    lane16 = lax.broadcasted_iota(jnp.int32, (1, 16), 1)
    pvec = jnp.zeros((1, 16), jnp.int32)
    zeros_shift = jnp.zeros((1, P - 1), jnp.int32)
    for b in range(B):
        ids = ids_ref[b:b + 1, :]
        m = mask_ref[b:b + 1, :]
        p = jnp.sum(jnp.where(ids == IMAGE_TOKEN, lane_s, 0))
        a_low = jnp.concatenate([m, zeros_shift], axis=1)
        a_high = jnp.concatenate([zeros_shift, m], axis=1)
        sel = jnp.where(lane_e < p, a_low,
                        jnp.where(lane_e < p + P, jnp.int32(1), a_high))
        cs = sel
        sh = 1
        while sh < E:
            cs = cs + jnp.concatenate(
                [jnp.zeros((1, sh), jnp.int32), cs[:, :E - sh]], axis=1)
            sh *= 2
        pos = cs - 1
        pos = jnp.where(sel == 0, 1, pos)
        outmask_ref[b:b + 1, :] = sel
        pos_ref[b:b + 1, :] = pos
        EP = -(-E // 8) * 8
        hb = b % 2
        de = hb * EP + lane_s + jnp.where(lane_s > p, P - 1, 0)
        de_ref[b:b + 1, :] = jnp.where(lane_s == p, hb * EP + (EP - 1), de)
        di_ref[b:b + 1, :] = hb * EP + p + lane_p
        pvec = jnp.where(lane16 == b, p, pvec)
    pvec_ref[...] = pvec


_SLOTS = 3


def _row_src(emb_hbm, img_hbm, b, r, p, *, S, P):
    return [
        (r < p, lambda: emb_hbm.at[pl.ds(b * S + r, 1)]),
        ((r >= p) & (r < p + P), lambda: img_hbm.at[pl.ds(b * P + r - p, 1)]),
        (r >= p + P, lambda: emb_hbm.at[pl.ds(b * S + r - (P - 1), 1)]),
    ]


def _row_copy_sync(emb_hbm, img_hbm, out_hbm, rowbuf, b, r, p, *, S, P, E):
    dst = out_hbm.at[pl.ds(b * E + r, 1)]
    for pred, src in _row_src(emb_hbm, img_hbm, b, r, p, S=S, P=P):
        @pl.when(pred)
        def _(src=src):
            pltpu.sync_copy(src(), rowbuf)
            pltpu.sync_copy(rowbuf, dst)


def _sc_copy_kernel(emb_hbm, img_hbm, pvec_hbm, out_hbm, buf, pbuf,
                    in_sem, out_sem, *, B, S, P, E, D):
    cid = lax.axis_index("core")
    sid = lax.axis_index("subcore")
    wid = cid * _NUM_SUBCORES + sid
    pltpu.sync_copy(pvec_hbm, pbuf)
    pvals = pbuf[...]
    NB = E // _BLK
    TAIL0 = NB * _BLK
    KMAX = (NB + _NW - 1) // _NW
    jobs = [(b, k) for b in range(B) for k in range(KMAX)]

    def drain_in(slot, pred):
        @pl.when(pred)
        def _():
            pltpu.make_async_copy(emb_hbm.at[pl.ds(0, _BLK)], buf.at[slot],
                                  in_sem.at[slot]).wait()

    def drain_out(slot, pred):
        @pl.when(pred)
        def _():
            pltpu.make_async_copy(buf.at[slot], out_hbm.at[pl.ds(0, _BLK)],
                                  out_sem.at[slot]).wait()

    pend_in = [None] * _SLOTS
    pend_out = [None] * _SLOTS

    def finish_prev(slot):
        pred, b, r0 = pend_in[slot]
        drain_in(slot, pred)

        @pl.when(pred)
        def _():
            pltpu.async_copy(buf.at[slot],
                             out_hbm.at[pl.ds(b * E + r0, _BLK)],
                             out_sem.at[slot])
        pend_out[slot] = pred
        pend_in[slot] = None

    for j, (b, k) in enumerate(jobs):
        slot = j % _SLOTS
        p = pvals[b]
        hi0 = p + P
        blk = k * _NW + wid
        pred = blk < NB
        r0 = blk * _BLK
        if pend_out[slot] is not None:
            drain_out(slot, pend_out[slot])
            pend_out[slot] = None

        emb_pure = (r0 + _BLK <= p) | (r0 >= hi0)
        emb_off = jnp.where(r0 + _BLK <= p, r0, r0 - (P - 1))
        img_pure = (r0 >= p) & (r0 + _BLK <= hi0)
        straddle = pred & (~emb_pure) & (~img_pure)

        @pl.when(pred & emb_pure)
        def _(b=b, emb_off=emb_off, slot=slot):
            pltpu.async_copy(emb_hbm.at[pl.ds(b * S + emb_off, _BLK)],
                             buf.at[slot], in_sem.at[slot])

        @pl.when(pred & img_pure)
        def _(b=b, r0=r0, p=p, slot=slot):
            pltpu.async_copy(img_hbm.at[pl.ds(b * P + r0 - p, _BLK)],
                             buf.at[slot], in_sem.at[slot])

        @pl.when(straddle)
        def _(b=b, r0=r0, p=p, slot=slot):
            @pl.loop(r0, r0 + _BLK)
            def _(r):
                for spred, src in _row_src(emb_hbm, img_hbm, b, r, p, S=S, P=P):
                    @pl.when(spred)
                    def _(src=src):
                        pltpu.async_copy(src(),
                                         buf.at[slot].at[pl.ds(r - r0, 1)],
                                         in_sem.at[slot])

        if pend_in[(j - 1) % _SLOTS] is not None and _SLOTS > 1:
            finish_prev((j - 1) % _SLOTS)
        pend_in[slot] = (pred, b, r0)

    if pend_in[(len(jobs) - 1) % _SLOTS] is not None:
        finish_prev((len(jobs) - 1) % _SLOTS)
    for slot in range(_SLOTS):
        if pend_out[slot] is not None:
            drain_out(slot, pend_out[slot])
            pend_out[slot] = None

    for b in range(B):
        @pl.when(wid == b)
        def _(b=b):
            p = pvals[b]

            @pl.loop(TAIL0, E)
            def _(r):
                _row_copy_sync(emb_hbm, img_hbm, out_hbm,
                               buf.at[0].at[pl.ds(0, 1)], b, r, p,
                               S=S, P=P, E=E)


_W = 8


def _sc_scatter_kernel(emb_hbm, img_hbm, de_hbm, di_hbm, out_hbm,
                       buf, ibuf, in_sem, idx_sem, out_sem,
                       *, b0, b1, S, P, E, D):
    cid = lax.axis_index("core")
    sid = lax.axis_index("subcore")
    wid = cid * _NUM_SUBCORES + sid

    jobs = []
    for b in range(b0, b1):
        for k in range((S // _W + _NW - 1) // _NW):
            jobs.append((False, b, k * _NW, S // _W))
        for k in range((P // _W + _NW - 1) // _NW):
            jobs.append((True, b, k * _NW, P // _W))

    pend = [None] * _SLOTS

    def drain_out(slot, pred):
        @pl.when(pred)
        def _():
            pltpu.make_async_copy(buf.at[slot], out_hbm.at[pl.ds(0, _W)],
                                  out_sem.at[slot]).wait()

    def finish_in(slot, pred, is_img, b, w0):
        @pl.when(pred)
        def _():
            pltpu.make_async_copy(emb_hbm.at[0, pl.ds(0, _W), :], buf.at[slot],
                                  in_sem.at[slot]).wait()
            pltpu.make_async_copy(de_hbm.at[pl.ds(0, _W)], ibuf.at[slot],
                                  idx_sem.at[slot]).wait()
            pltpu.async_copy(buf.at[slot], out_hbm.at[ibuf.at[slot]],
                             out_sem.at[slot])

    prev = None
    for j, (is_img, b, kbase, nwin) in enumerate(jobs):
        slot = j % _SLOTS
        w = kbase + wid
        pred = w < nwin
        w0 = w * _W
        if pend[slot] is not None:
            drain_out(slot, pend[slot])
            pend[slot] = None

        @pl.when(pred)
        def _(is_img=is_img, b=b, w0=w0, slot=slot):
            if is_img:
                pltpu.async_copy(img_hbm.at[b, pl.ds(w0, _W), :],
                                 buf.at[slot], in_sem.at[slot])
                pltpu.async_copy(di_hbm.at[pl.ds(b * P + w0, _W)],
                                 ibuf.at[slot], idx_sem.at[slot])
            else:
                pltpu.async_copy(emb_hbm.at[b, pl.ds(w0, _W), :],
                                 buf.at[slot], in_sem.at[slot])
                pltpu.async_copy(de_hbm.at[pl.ds(b * S + w0, _W)],
                                 ibuf.at[slot], idx_sem.at[slot])

        if prev is not None:
            finish_in(*prev)
            pend[prev[0]] = prev[1]
        prev = (slot, pred, is_img, b, w0)
    if prev is not None:
        finish_in(*prev)
        pend[prev[0]] = prev[1]
    for slot in range(_SLOTS):
        if pend[slot] is not None:
            drain_out(slot, pend[slot])
            pend[slot] = None


def _sc_scatter(inputs_embeds, image_features, de, di, B, S, P, E, D):
    EP = -(-E // 8) * 8
    i32 = jnp.int32
    mesh = plsc.VectorSubcoreMesh(core_axis_name="core",
                                  subcore_axis_name="subcore")
    HB = B // 2
    halves = []
    for b0 in (0, HB):
        sc_fn = pl.kernel(
            functools.partial(_sc_scatter_kernel, b0=b0, b1=b0 + HB,
                              S=S, P=P, E=E, D=D),
            out_type=jax.ShapeDtypeStruct((HB * EP, D), inputs_embeds.dtype),
            mesh=mesh,
            scratch_types=[
                pltpu.VMEM((_SLOTS, _W, D), inputs_embeds.dtype),
                pltpu.VMEM((_SLOTS, _W), i32),
                pltpu.SemaphoreType.DMA((_SLOTS,)),
                pltpu.SemaphoreType.DMA((_SLOTS,)),
                pltpu.SemaphoreType.DMA((_SLOTS,)),
            ],
        )
        out = sc_fn(inputs_embeds, image_features,
                    de.reshape(-1), di.reshape(-1))
        halves.append(out.reshape(HB, EP, D)[:, :E, :])
    return jnp.concatenate(halves, axis=0)


_R = 128


def _tc_merge_kernel(p_ref, emb_a_ref, emb_b_ref,
                     img_a_ref, img_b_ref, out_ref, *, S, P, E, R, JP):
    b = pl.program_id(0)
    j = pl.program_id(1)
    r0 = j * R
    p = p_ref[b]
    w0 = jnp.where(r0 < p, r0, r0 - (P - 1))
    qe = jnp.clip(w0 // R, 0, S // R - 1)
    te = w0 - qe * R
    ch2 = jnp.concatenate([emb_a_ref[0], emb_b_ref[0]], axis=0)
    ch = pltpu.roll(ch2, -te, 0)[:R]
    qi = jnp.clip((r0 - p) // R, 0, JP - 1)
    ti = r0 - p - qi * R
    ci2 = jnp.concatenate([img_a_ref[0], img_b_ref[0]], axis=0)
    ci = pltpu.roll(ci2, -ti, 0)[:R]
    ri = r0 + lax.broadcasted_iota(jnp.int32, (R, 1), 0)
    out_ref[0] = jnp.where((ri >= p) & (ri < p + P), ci, ch)


def _tc_merge(inputs_embeds, image_features, pvec, B, S, P, E, D):
    R = _R
    JE = (E + R - 1) // R
    JP = (P + R - 1) // R
    JS = S // R

    def emb_q(j, pref, b):
        r0 = j * R
        w0 = jnp.where(r0 < pref[b], r0, r0 - (P - 1))
        return jnp.clip(w0 // R, 0, JS - 1)

    def im_emb_a(b, j, pref):
        return (b, emb_q(j, pref, b), 0)

    def im_emb_b(b, j, pref):
        return (b, jnp.minimum(emb_q(j, pref, b) + 1, JS - 1), 0)

    def im_img_a(b, j, pref):
        return (b, jnp.clip((j * R - pref[b]) // R, 0, JP - 1), 0)

    def im_img_b(b, j, pref):
        return (b, jnp.clip((j * R - pref[b]) // R + 1, 0, JP - 1), 0)

    grid_spec = pltpu.PrefetchScalarGridSpec(
        num_scalar_prefetch=1,
        grid=(B, JE),
        in_specs=[
            pl.BlockSpec((1, R, D), im_emb_a),
            pl.BlockSpec((1, R, D), im_emb_b),
            pl.BlockSpec((1, R, D), im_img_a),
            pl.BlockSpec((1, R, D), im_img_b),
        ],
        out_specs=pl.BlockSpec((1, R, D), lambda b, j, pref: (b, j, 0)),
    )
    return pl.pallas_call(
        functools.partial(_tc_merge_kernel, S=S, P=P, E=E, R=R, JP=JP),
        grid_spec=grid_spec,
        out_shape=jax.ShapeDtypeStruct((B, E, D), inputs_embeds.dtype),
        compiler_params=pltpu.CompilerParams(
            dimension_semantics=("parallel", "arbitrary")),
    )(pvec, inputs_embeds, inputs_embeds,
      image_features, image_features)


def kernel(inputs_embeds, image_features, input_ids, attention_mask):
    B, S, D = inputs_embeds.shape
    P = image_features.shape[1]
    E = S + P - 1

    i32 = jnp.int32
    outmask, pos, pvec, de, di = pl.pallas_call(
        functools.partial(_mask_pos_kernel, S=S, P=P, E=E),
        out_shape=[
            jax.ShapeDtypeStruct((B, E), i32),
            jax.ShapeDtypeStruct((B, E), i32),
            jax.ShapeDtypeStruct((1, 16), i32),
            jax.ShapeDtypeStruct((B, S), i32),
            jax.ShapeDtypeStruct((B, P), i32),
        ],
    )(input_ids.astype(i32), attention_mask.astype(i32))

    del pvec
    final = _sc_scatter(inputs_embeds, image_features, de, di,
                        B, S, P, E, D)
    return (final, outmask.astype(attention_mask.dtype), pos)


# speedup vs baseline: 1.4620x; 1.4620x over previous
import functools

import jax
import jax.numpy as jnp
from jax import lax
from jax.experimental import pallas as pl
from jax.experimental.pallas import tpu as pltpu
from jax.experimental.pallas import tpu_sc as plsc

IMAGE_TOKEN = 32000
_NUM_CORES = 2
_NUM_SUBCORES = 16
_NW = _NUM_CORES * _NUM_SUBCORES
_SLOTS = 3
_W = 8


def _mask_pos_kernel(ids_ref, mask_ref, outmask_ref, pos_ref,
                     de_ref, di_ref, *, S, P, E):
    B = ids_ref.shape[0]
    EP = -(-E // 8) * 8
    lane_e = lax.broadcasted_iota(jnp.int32, (1, E), 1)
    lane_s = lax.broadcasted_iota(jnp.int32, (1, S), 1)
    lane_p = lax.broadcasted_iota(jnp.int32, (1, P), 1)
    zeros_shift = jnp.zeros((1, P - 1), jnp.int32)
    for b in range(B):
        ids = ids_ref[b:b + 1, :]
        m = mask_ref[b:b + 1, :]
        p = jnp.sum(jnp.where(ids == IMAGE_TOKEN, lane_s, 0))
        a_low = jnp.concatenate([m, zeros_shift], axis=1)
        a_high = jnp.concatenate([zeros_shift, m], axis=1)
        sel = jnp.where(lane_e < p, a_low,
                        jnp.where(lane_e < p + P, jnp.int32(1), a_high))
        cs = sel
        sh = 1
        while sh < E:
            cs = cs + jnp.concatenate(
                [jnp.zeros((1, sh), jnp.int32), cs[:, :E - sh]], axis=1)
            sh *= 2
        pos = cs - 1
        pos = jnp.where(sel == 0, 1, pos)
        outmask_ref[b:b + 1, :] = sel
        pos_ref[b:b + 1, :] = pos
        de = b * EP + lane_s + jnp.where(lane_s > p, P - 1, 0)
        de_ref[b:b + 1, :] = jnp.where(lane_s == p, b * EP + (EP - 1), de)
        di_ref[b:b + 1, :] = b * EP + p + lane_p


def _sc_scatter_kernel(emb_hbm, img_hbm, de_hbm, di_hbm, out_hbm,
                       buf, ibuf, in_sem, idx_sem, out_sem,
                       *, B, S, P, E, D):
    cid = lax.axis_index("core")
    sid = lax.axis_index("subcore")
    wid = cid * _NUM_SUBCORES + sid

    jobs = []
    for b in range(B):
        for k in range((S // _W + _NW - 1) // _NW):
            jobs.append((False, b, k * _NW, S // _W))
        for k in range((P // _W + _NW - 1) // _NW):
            jobs.append((True, b, k * _NW, P // _W))

    pend = [None] * _SLOTS

    def drain_out(slot, pred):
        @pl.when(pred)
        def _():
            pltpu.make_async_copy(buf.at[slot], out_hbm.at[pl.ds(0, _W)],
                                  out_sem.at[slot]).wait()

    def finish_in(slot, pred, is_img, b, w0):
        @pl.when(pred)
        def _():
            pltpu.make_async_copy(emb_hbm.at[0, pl.ds(0, _W), :], buf.at[slot],
                                  in_sem.at[slot]).wait()
            pltpu.make_async_copy(de_hbm.at[pl.ds(0, _W)], ibuf.at[slot],
                                  idx_sem.at[slot]).wait()
            pltpu.async_copy(buf.at[slot], out_hbm.at[ibuf.at[slot]],
                             out_sem.at[slot])

    prev = None
    for j, (is_img, b, kbase, nwin) in enumerate(jobs):
        slot = j % _SLOTS
        w = kbase + wid
        pred = w < nwin
        w0 = w * _W
        if pend[slot] is not None:
            drain_out(slot, pend[slot])
            pend[slot] = None

        @pl.when(pred)
        def _(is_img=is_img, b=b, w0=w0, slot=slot):
            if is_img:
                pltpu.async_copy(img_hbm.at[b, pl.ds(w0, _W), :],
                                 buf.at[slot], in_sem.at[slot])
                pltpu.async_copy(di_hbm.at[pl.ds(b * P + w0, _W)],
                                 ibuf.at[slot], idx_sem.at[slot])
            else:
                pltpu.async_copy(emb_hbm.at[b, pl.ds(w0, _W), :],
                                 buf.at[slot], in_sem.at[slot])
                pltpu.async_copy(de_hbm.at[pl.ds(b * S + w0, _W)],
                                 ibuf.at[slot], idx_sem.at[slot])

        if prev is not None:
            finish_in(*prev)
            pend[prev[0]] = prev[1]
        prev = (slot, pred, is_img, b, w0)
    if prev is not None:
        finish_in(*prev)
        pend[prev[0]] = prev[1]
    for slot in range(_SLOTS):
        if pend[slot] is not None:
            drain_out(slot, pend[slot])
            pend[slot] = None


def _sc_scatter(inputs_embeds, image_features, de, di, B, S, P, E, D):
    EP = -(-E // 8) * 8
    i32 = jnp.int32
    mesh = plsc.VectorSubcoreMesh(core_axis_name="core",
                                  subcore_axis_name="subcore")
    sc_fn = pl.kernel(
        functools.partial(_sc_scatter_kernel, B=B, S=S, P=P, E=E, D=D),
        out_type=jax.ShapeDtypeStruct((B * EP, D), inputs_embeds.dtype),
        mesh=mesh,
        scratch_types=[
            pltpu.VMEM((_SLOTS, _W, D), inputs_embeds.dtype),
            pltpu.VMEM((_SLOTS, _W), i32),
            pltpu.SemaphoreType.DMA((_SLOTS,)),
            pltpu.SemaphoreType.DMA((_SLOTS,)),
            pltpu.SemaphoreType.DMA((_SLOTS,)),
        ],
    )
    out = sc_fn(inputs_embeds, image_features,
                de.reshape(-1), di.reshape(-1))
    return out.reshape(B, EP, D)[:, :E, :]


def kernel(inputs_embeds, image_features, input_ids, attention_mask):
    B, S, D = inputs_embeds.shape
    P = image_features.shape[1]
    E = S + P - 1

    i32 = jnp.int32
    outmask, pos, de, di = pl.pallas_call(
        functools.partial(_mask_pos_kernel, S=S, P=P, E=E),
        out_shape=[
            jax.ShapeDtypeStruct((B, E), i32),
            jax.ShapeDtypeStruct((B, E), i32),
            jax.ShapeDtypeStruct((B, S), i32),
            jax.ShapeDtypeStruct((B, P), i32),
        ],
    )(input_ids.astype(i32), attention_mask.astype(i32))

    final = _sc_scatter(inputs_embeds, image_features, de, di,
                        B, S, P, E, D)
    return (final, outmask.astype(attention_mask.dtype), pos)
